# baseline (speedup 1.0000x reference)
import jax
import jax.numpy as jnp
from jax import lax
from jax.experimental import pallas as pl
from jax.experimental.pallas import tpu as pltpu
from jax.experimental.pallas import tpu_sc as plsc

B, C, H, W, K = 64, 2, 128, 128, 128
HW = H * W
CHW = C * HW
L = 16
NS = 16
BPW = B // NS


def _stage1_body(outflat, maski, ind, tgt, out,
                 indall, maskall, tgtall, idxs, preds, partv,
                 sem_i, sem_m, sem_t, sem_gs):
    cid = lax.axis_index("c")
    sid = lax.axis_index("s")

    @pl.when(cid == 0)
    def _work():
        b0 = sid * BPW
        st0 = pltpu.async_copy(ind.at[sid], indall, sem_i)
        st1 = pltpu.async_copy(maski.at[sid], maskall, sem_m)
        st2 = pltpu.async_copy(tgt.at[sid], tgtall, sem_t)
        st0.wait()
        gathers = []
        for bl in range(BPW):
            base = (b0 + bl) * CHW
            for j in range(8):
                sl = pl.ds(L * j, L)
                v = indall[pl.ds(K * bl + L * j, L)] + base
                idxs[2 * bl][sl] = v
                idxs[2 * bl + 1][sl] = v + HW
            for r in (2 * bl, 2 * bl + 1):
                gathers.append(
                    pltpu.async_copy(outflat.at[idxs[r]], preds[r], sem_gs[r]))
        st1.wait()
        st2.wait()
        for cp in gathers:
            cp.wait()
        acc = jnp.zeros((L,), jnp.float32)
        msum = jnp.zeros((L,), jnp.float32)
        for bl in range(BPW):
            for j in range(8):
                sl = pl.ds(L * j, L)
                mk = maskall[pl.ds(K * bl + L * j, L)].astype(jnp.float32)
                d0 = jnp.abs(preds[2 * bl][sl]
                             - tgtall[pl.ds(2 * K * bl + L * j, L)])
                d1 = jnp.abs(preds[2 * bl + 1][sl]
                             - tgtall[pl.ds(2 * K * bl + K + L * j, L)])
                acc = acc + (d0 + d1) * mk
                msum = msum + mk
        partv[pl.ds(0, L)] = acc
        partv[pl.ds(L, L)] = msum
        pltpu.sync_copy(partv, out.at[sid])


def _stage2_body(parts, out, redv, outv, sem_r):
    cid = lax.axis_index("c")
    sid = lax.axis_index("s")

    @pl.when(jnp.logical_and(cid == 0, sid == 0))
    def _reduce():
        pltpu.async_copy(parts, redv, sem_r).wait()
        ta = jnp.zeros((L,), jnp.float32)
        tm = jnp.zeros((L,), jnp.float32)
        for t in range(NS):
            ta = ta + redv[t, pl.ds(0, L)]
            tm = tm + redv[t, pl.ds(L, L)]
        num = jnp.float32(0.0)
        den = jnp.float32(0.0)
        for i in range(L):
            num = num + ta[i]
            den = den + tm[i]
        den = den * jnp.float32(C) + jnp.float32(1e-4)
        numv = jnp.full((L,), num, jnp.float32)
        denv = jnp.full((L,), den, jnp.float32)
        outv[...] = numv / denv
        pltpu.sync_copy(outv, out)


_MESH = plsc.VectorSubcoreMesh(core_axis_name="c", subcore_axis_name="s")

_stage1 = pl.kernel(
    _stage1_body,
    out_type=jax.ShapeDtypeStruct((NS, 2 * L), jnp.float32),
    mesh=_MESH,
    scratch_types=[
        pltpu.VMEM((BPW * K,), jnp.int32),
        pltpu.VMEM((BPW * K,), jnp.int32),
        pltpu.VMEM((2 * BPW * K,), jnp.float32),
        [pltpu.VMEM((K,), jnp.int32) for _ in range(2 * BPW)],
        [pltpu.VMEM((K,), jnp.float32) for _ in range(2 * BPW)],
        pltpu.VMEM((2 * L,), jnp.float32),
        pltpu.SemaphoreType.DMA,
        pltpu.SemaphoreType.DMA,
        pltpu.SemaphoreType.DMA,
        [pltpu.SemaphoreType.DMA for _ in range(2 * BPW)],
    ],
)

_stage2 = pl.kernel(
    _stage2_body,
    out_type=jax.ShapeDtypeStruct((L,), jnp.float32),
    mesh=_MESH,
    scratch_types=[
        pltpu.VMEM((NS, 2 * L), jnp.float32),
        pltpu.VMEM((L,), jnp.float32),
        pltpu.SemaphoreType.DMA,
    ],
)


def kernel(output, mask, ind, target):
    outflat = output.reshape(B * C * HW)
    mask32 = mask.astype(jnp.int32).reshape(NS, BPW * K)
    ind32 = ind.astype(jnp.int32).reshape(NS, BPW * K)
    tgt = jnp.transpose(target, (0, 2, 1)).reshape(NS, 2 * BPW * K)
    parts = _stage1(outflat, mask32, ind32, tgt)
    res = _stage2(parts)
    return res[0]


# speedup vs baseline: 1.0566x; 1.0566x over previous
import jax
import jax.numpy as jnp
from jax import lax
from jax.experimental import pallas as pl
from jax.experimental.pallas import tpu as pltpu
from jax.experimental.pallas import tpu_sc as plsc

B, C, H, W, K = 64, 2, 128, 128, 128
HW = H * W
CHW = C * HW
L = 16
NC = 2
NS = 16
NW = NC * NS
BPW = B // NW


def _stage1_body(outflat, maski, ind, tgt, out,
                 indr, maskr, tgtall, idxs, preds, partv,
                 sem_is, sem_ms, sem_t, sem_gs):
    cid = lax.axis_index("c")
    sid = lax.axis_index("s")
    wid = sid * NC + cid
    b0 = wid * BPW
    sti = [pltpu.async_copy(ind.at[b0 + bl], indr[bl], sem_is[bl])
           for bl in range(BPW)]
    stm = [pltpu.async_copy(maski.at[b0 + bl], maskr[bl], sem_ms[bl])
           for bl in range(BPW)]
    stt = pltpu.async_copy(tgt.at[wid], tgtall, sem_t)
    gathers = []
    for bl in range(BPW):
        sti[bl].wait()
        base = (b0 + bl) * CHW
        for j in range(8):
            sl = pl.ds(L * j, L)
            v = indr[bl][sl] + base
            idxs[2 * bl][sl] = v
            idxs[2 * bl + 1][sl] = v + HW
        for r in (2 * bl, 2 * bl + 1):
            gathers.append(
                pltpu.async_copy(outflat.at[idxs[r]], preds[r], sem_gs[r]))
    for st in stm:
        st.wait()
    stt.wait()
    for cp in gathers:
        cp.wait()
    acc = jnp.zeros((L,), jnp.float32)
    msum = jnp.zeros((L,), jnp.float32)
    for bl in range(BPW):
        for j in range(8):
            sl = pl.ds(L * j, L)
            mk = maskr[bl][sl].astype(jnp.float32)
            d0 = jnp.abs(preds[2 * bl][sl]
                         - tgtall[pl.ds(2 * K * bl + L * j, L)])
            d1 = jnp.abs(preds[2 * bl + 1][sl]
                         - tgtall[pl.ds(2 * K * bl + K + L * j, L)])
            acc = acc + (d0 + d1) * mk
            msum = msum + mk
    partv[pl.ds(0, L)] = acc
    partv[pl.ds(L, L)] = msum
    pltpu.sync_copy(partv, out.at[wid])


def _stage2_body(parts, out, redv, outv, sem_r):
    cid = lax.axis_index("c")
    sid = lax.axis_index("s")

    @pl.when(jnp.logical_and(cid == 0, sid == 0))
    def _reduce():
        pltpu.async_copy(parts, redv, sem_r).wait()
        ta = jnp.zeros((L,), jnp.float32)
        tm = jnp.zeros((L,), jnp.float32)
        for t in range(NW):
            ta = ta + redv[t, pl.ds(0, L)]
            tm = tm + redv[t, pl.ds(L, L)]
        num = jnp.float32(0.0)
        den = jnp.float32(0.0)
        for i in range(L):
            num = num + ta[i]
            den = den + tm[i]
        den = den * jnp.float32(C) + jnp.float32(1e-4)
        numv = jnp.full((L,), num, jnp.float32)
        denv = jnp.full((L,), den, jnp.float32)
        outv[...] = numv / denv
        pltpu.sync_copy(outv, out)


_MESH = plsc.VectorSubcoreMesh(core_axis_name="c", subcore_axis_name="s")

_stage1 = pl.kernel(
    _stage1_body,
    out_type=jax.ShapeDtypeStruct((NW, 2 * L), jnp.float32),
    mesh=_MESH,
    scratch_types=[
        [pltpu.VMEM((K,), jnp.int32) for _ in range(BPW)],
        [pltpu.VMEM((K,), jnp.int32) for _ in range(BPW)],
        pltpu.VMEM((C * BPW * K,), jnp.float32),
        [pltpu.VMEM((K,), jnp.int32) for _ in range(2 * BPW)],
        [pltpu.VMEM((K,), jnp.float32) for _ in range(2 * BPW)],
        pltpu.VMEM((2 * L,), jnp.float32),
        [pltpu.SemaphoreType.DMA for _ in range(BPW)],
        [pltpu.SemaphoreType.DMA for _ in range(BPW)],
        pltpu.SemaphoreType.DMA,
        [pltpu.SemaphoreType.DMA for _ in range(2 * BPW)],
    ],
)

_stage2 = pl.kernel(
    _stage2_body,
    out_type=jax.ShapeDtypeStruct((L,), jnp.float32),
    mesh=_MESH,
    scratch_types=[
        pltpu.VMEM((NW, 2 * L), jnp.float32),
        pltpu.VMEM((L,), jnp.float32),
        pltpu.SemaphoreType.DMA,
    ],
)


def kernel(output, mask, ind, target):
    outflat = output.reshape(B * C * HW)
    tgt = jnp.transpose(target, (0, 2, 1)).reshape(NW, C * BPW * K)
    parts = _stage1(outflat, mask.astype(jnp.int32), ind.astype(jnp.int32),
                    tgt)
    res = _stage2(parts)
    return res[0]


# speedup vs baseline: 1.1022x; 1.0431x over previous
import jax
import jax.numpy as jnp
from jax import lax
from jax.experimental import pallas as pl
from jax.experimental.pallas import tpu as pltpu
from jax.experimental.pallas import tpu_sc as plsc

B, C, H, W, K = 64, 2, 128, 128, 128
HW = H * W
CHW = C * HW
L = 16
NC = 2
NS = 16
NW = NC * NS
BPW = B // NW


def _stage1_body(outflat, maski, ind, tgt, out,
                 indr, maskr, tgtall, idxs, preds, partv,
                 sem_is, sem_ms, sem_t, sem_gs):
    cid = lax.axis_index("c")
    sid = lax.axis_index("s")
    wid = sid * NC + cid
    b0 = wid * BPW
    sti = [pltpu.async_copy(ind.at[b0 + bl], indr[bl], sem_is[bl])
           for bl in range(BPW)]
    stm = [pltpu.async_copy(maski.at[b0 + bl], maskr[bl], sem_ms[bl])
           for bl in range(BPW)]
    stt = pltpu.async_copy(tgt.at[wid], tgtall, sem_t)
    gathers = []
    for bl in range(BPW):
        sti[bl].wait()
        base = (b0 + bl) * CHW
        for j in range(8):
            sl = pl.ds(L * j, L)
            v = indr[bl][sl] + base
            idxs[2 * bl][sl] = v
            idxs[2 * bl + 1][sl] = v + HW
        for r in (2 * bl, 2 * bl + 1):
            gathers.append(
                pltpu.async_copy(outflat.at[idxs[r]], preds[r], sem_gs[r]))
    for st in stm:
        st.wait()
    stt.wait()
    for cp in gathers:
        cp.wait()
    acc = jnp.zeros((L,), jnp.float32)
    msum = jnp.zeros((L,), jnp.float32)
    for bl in range(BPW):
        for j in range(8):
            sl = pl.ds(L * j, L)
            mk = maskr[bl][sl].astype(jnp.float32)
            d0 = jnp.abs(preds[2 * bl][sl]
                         - tgtall[pl.ds(2 * K * bl + L * j, L)])
            d1 = jnp.abs(preds[2 * bl + 1][sl]
                         - tgtall[pl.ds(2 * K * bl + K + L * j, L)])
            acc = acc + (d0 + d1) * mk
            msum = msum + mk
    partv[pl.ds(0, L)] = acc
    partv[pl.ds(L, L)] = msum
    pltpu.sync_copy(partv, out.at[wid])


def _stage2_body(parts, out, redv, outv, sem_r):
    cid = lax.axis_index("c")
    sid = lax.axis_index("s")

    @pl.when(jnp.logical_and(cid == 0, sid == 0))
    def _reduce():
        pltpu.async_copy(parts, redv, sem_r).wait()
        ta = jnp.zeros((L,), jnp.float32)
        tm = jnp.zeros((L,), jnp.float32)
        for t in range(NW):
            ta = ta + redv[t, pl.ds(0, L)]
            tm = tm + redv[t, pl.ds(L, L)]
        num = jnp.float32(0.0)
        den = jnp.float32(0.0)
        for i in range(L):
            num = num + ta[i]
            den = den + tm[i]
        den = den * jnp.float32(C) + jnp.float32(1e-4)
        numv = jnp.full((L,), num, jnp.float32)
        denv = jnp.full((L,), den, jnp.float32)
        outv[...] = numv / denv
        pltpu.sync_copy(outv, out)


_MESH = plsc.VectorSubcoreMesh(core_axis_name="c", subcore_axis_name="s")

_stage1 = pl.kernel(
    _stage1_body,
    out_type=jax.ShapeDtypeStruct((NW, 2 * L), jnp.float32),
    mesh=_MESH,
    scratch_types=[
        [pltpu.VMEM((K,), jnp.int32) for _ in range(BPW)],
        [pltpu.VMEM((K,), jnp.int32) for _ in range(BPW)],
        pltpu.VMEM((C * BPW * K,), jnp.float32),
        [pltpu.VMEM((K,), jnp.int32) for _ in range(2 * BPW)],
        [pltpu.VMEM((K,), jnp.float32) for _ in range(2 * BPW)],
        pltpu.VMEM((2 * L,), jnp.float32),
        [pltpu.SemaphoreType.DMA for _ in range(BPW)],
        [pltpu.SemaphoreType.DMA for _ in range(BPW)],
        pltpu.SemaphoreType.DMA,
        [pltpu.SemaphoreType.DMA for _ in range(2 * BPW)],
    ],
)

_stage2 = pl.kernel(
    _stage2_body,
    out_type=jax.ShapeDtypeStruct((L,), jnp.float32),
    mesh=plsc.VectorSubcoreMesh(core_axis_name="c", subcore_axis_name="s",
                                num_cores=1),
    scratch_types=[
        pltpu.VMEM((NW, 2 * L), jnp.float32),
        pltpu.VMEM((L,), jnp.float32),
        pltpu.SemaphoreType.DMA,
    ],
)


def kernel(output, mask, ind, target):
    outflat = output.reshape(B * C * HW)
    tgt = jnp.transpose(target, (0, 2, 1)).reshape(NW, C * BPW * K)
    parts = _stage1(outflat, mask.astype(jnp.int32), ind.astype(jnp.int32),
                    tgt)
    res = _stage2(parts)
    return res[0]
